# baseline (speedup 1.0000x reference)
import functools

import jax
import jax.numpy as jnp
from jax import lax
from jax.experimental import pallas as pl
from jax.experimental.pallas import tpu as pltpu

_K = 16
_ITERS = 3


def _kmedoids_body(x_ref, w_ref, out_ref, xt3_ref):
    xb = x_ref[0]
    wb = w_ref[0]
    F, S = xb.shape
    k = _K
    G = F // 8

    w_tok = jnp.mean(wb, axis=0, keepdims=True)

    for g in range(G):
        xt3_ref[g] = xb[8 * g : 8 * g + 8, :].T

    def g_body(g, dacc):
        xg = x_ref[0, pl.ds(g * 8, 8), :]
        xtg = xt3_ref[g]
        for u in range(8):
            col = xtg[:, u : u + 1]
            row = xg[u : u + 1, :]
            dacc = dacc + jnp.abs(col - row)
        return dacc

    d = lax.fori_loop(0, G, g_body, jnp.zeros((S, S), jnp.float32))

    lane_iota = lax.broadcasted_iota(jnp.int32, (1, S), 1)
    k_iota = lax.broadcasted_iota(jnp.int32, (1, k), 1)

    def topk_body(c, carry):
        w_cur, ctr = carry
        m = jnp.max(w_cur)
        idx = jnp.min(jnp.where(w_cur == m, lane_iota, S))
        ctr = jnp.where(k_iota == c, idx, ctr)
        w_cur = jnp.where(lane_iota == idx, -jnp.inf, w_cur)
        return w_cur, ctr

    _, ctr = lax.fori_loop(
        0, k, topk_body, (w_tok, jnp.zeros((1, k), jnp.int32))
    )

    s_iota_col = lax.broadcasted_iota(jnp.int32, (S, 1), 0)
    k_iota_row = lax.broadcasted_iota(jnp.int32, (S, k), 1)
    s_iota_sk = lax.broadcasted_iota(jnp.int32, (S, k), 0)
    w_col = w_tok.T

    def iter_body(_, ctr):
        oh_ctr = (s_iota_col == ctr).astype(jnp.float32)
        i2c = lax.dot(d, oh_ctr, preferred_element_type=jnp.float32)
        mn = jnp.min(i2c, axis=1, keepdims=True)
        assign = jnp.min(
            jnp.where(i2c == mn, k_iota_row, k), axis=1, keepdims=True
        )
        oh_a = (assign == k_iota_row).astype(jnp.float32)
        cost = lax.dot(
            d, oh_a * w_col, preferred_element_type=jnp.float32
        )
        cost = jnp.where(oh_a > 0, cost, jnp.inf)
        mnc = jnp.min(cost, axis=0, keepdims=True)
        ctr = jnp.min(
            jnp.where(cost == mnc, s_iota_sk, S), axis=0, keepdims=True
        )
        return ctr

    ctr = lax.fori_loop(0, _ITERS, iter_body, ctr)

    oh_ctr = (s_iota_col == ctr).astype(jnp.float32)
    out_ref[0] = lax.dot(xb, oh_ctr, preferred_element_type=jnp.float32)


@jax.jit
def kernel(x, w):
    B, F, S = x.shape
    k = _K
    if k >= S:
        return x
    return pl.pallas_call(
        _kmedoids_body,
        grid=(B,),
        in_specs=[
            pl.BlockSpec((1, F, S), lambda b: (b, 0, 0)),
            pl.BlockSpec((1, S, S), lambda b: (b, 0, 0)),
        ],
        out_specs=pl.BlockSpec((1, F, k), lambda b: (b, 0, 0)),
        out_shape=jax.ShapeDtypeStruct((B, F, k), x.dtype),
        scratch_shapes=[pltpu.VMEM((F // 8, S, 8), jnp.float32)],
    )(x, w)
